# Initial kernel scaffold; baseline (speedup 1.0000x reference)
import jax
import jax.numpy as jnp
from jax.experimental import pallas as pl


def kernel(u, a_true, noise_r, noise_i, ebno_db, W1, b1, W2x, b2x, W2p, b2p):
    raise NotImplementedError("write your pallas kernel here")


# speedup vs baseline: 3.1838x; 3.1838x over previous
import numpy as np
import jax
import jax.numpy as jnp
from jax.experimental import pallas as pl

_K = 128
_N = 256
_HIDDEN = 512
_B = 1024
_THRESH = 0.5
_RATE = _K / _N


def _build_info_mask():
    m = int(np.log2(_N))
    z = np.array([0.5], dtype=np.float64)
    for _ in range(m):
        z = np.concatenate([2.0 * z - z * z, z * z])
    order = np.argsort(z, kind="stable")
    mask = np.zeros(_N, dtype=bool)
    mask[order[:_K]] = True
    return mask


_INFO_MASK = _build_info_mask()
_FROZEN = ~_INFO_MASK
_INFO_IDX = np.where(_INFO_MASK)[0]


def _encode_rows(mat):
    n = mat.shape[1]
    if n == 1:
        return mat
    h = n // 2
    a = _encode_rows(mat[:, :h])
    b = _encode_rows(mat[:, h:])
    return np.concatenate([a ^ b, b], axis=1)


_G = _encode_rows(np.eye(_N, dtype=np.int64))
_G_INFO = _G[_INFO_IDX, :].astype(np.float32)
_G_UHAT = _G[:, _INFO_IDX].astype(np.float32)


def _mod2(v):
    return v - 2.0 * jnp.floor(v * 0.5)


def _dot_t(a, b):
    return jax.lax.dot_general(a, b, (((0,), (0,)), ((), ())),
                               preferred_element_type=jnp.float32)


def _decode_x(llr, fr):
    if fr.all():
        return jnp.zeros_like(llr)
    if not fr.any():
        return (llr < 0.0).astype(llr.dtype)
    h = fr.shape[0] // 2
    la, lb = llr[:h], llr[h:]
    lf, rf = fr[:h], fr[h:]
    if lf.all():
        x2 = _decode_x(la + lb, rf)
        return jnp.concatenate([x2, x2], axis=0)
    l1 = jnp.sign(la) * jnp.sign(lb) * jnp.minimum(jnp.abs(la), jnp.abs(lb))
    x1 = _decode_x(l1, lf)
    l2 = lb + (1.0 - 2.0 * x1) * la
    x2 = _decode_x(l2, rf)
    return jnp.concatenate([x1 + x2 - 2.0 * x1 * x2, x2], axis=0)


def _fused_body(u_ref, a_ref, nr_ref, ni_ref, no_ref, s_ref,
                w1_ref, b1_ref, w2x_ref, b2x_ref, w2p_ref, b2p_ref,
                gi_ref, gu_ref,
                ct_ref, uh_ref, ch_ref, pa_ref, ah_ref):
    ct = jax.lax.dot_general(gi_ref[...], u_ref[...], (((0,), (1,)), ((), ())),
                             preferred_element_type=jnp.float32)
    ct = _mod2(ct)
    ct_ref[...] = ct

    s = s_ref[0, 0]
    y_r = (1.0 - 2.0 * ct) * a_ref[...] + nr_ref[...] * s
    y_i = ni_ref[...] * s

    y_ri = jnp.concatenate([y_r, y_i], axis=0)
    h = jnp.maximum(_dot_t(w1_ref[...], y_ri) + b1_ref[...], 0.0)
    y_hat_r = _dot_t(w2x_ref[...], h) + b2x_ref[...]
    p = jax.nn.sigmoid(_dot_t(w2p_ref[...], h) + b2p_ref[...])
    pa_ref[...] = p
    ah = (p > _THRESH).astype(jnp.float32)
    ah_ref[...] = ah

    llr = (4.0 / no_ref[0, 0]) * y_hat_r
    x_root = _decode_x(llr, _FROZEN)
    ch_ref[...] = x_root * ah
    uh = _mod2(_dot_t(gu_ref[...], x_root))
    uh_ref[...] = uh * ah


def kernel(u, a_true, noise_r, noise_i, ebno_db, W1, b1, W2x, b2x, W2p, b2p):
    no = 1.0 / (jnp.power(10.0, ebno_db / 10.0) * _RATE)
    s = jnp.sqrt(no / 2.0)
    out_shapes = [
        jax.ShapeDtypeStruct((_N, _B), jnp.float32),
        jax.ShapeDtypeStruct((_K, _B), jnp.float32),
        jax.ShapeDtypeStruct((_N, _B), jnp.float32),
        jax.ShapeDtypeStruct((1, _B), jnp.float32),
        jax.ShapeDtypeStruct((1, _B), jnp.float32),
    ]
    ct_t, uh_t, ch_t, pa_t, ah_t = pl.pallas_call(
        _fused_body,
        out_shape=out_shapes,
    )(
        u,
        a_true.reshape(1, _B),
        noise_r.T,
        noise_i.T,
        no.reshape(1, 1),
        s.reshape(1, 1),
        W1,
        b1.reshape(_HIDDEN, 1),
        W2x[:, :_N],
        b2x[:_N].reshape(_N, 1),
        W2p,
        b2p.reshape(1, 1),
        jnp.asarray(_G_INFO),
        jnp.asarray(_G_UHAT),
    )
    return (u, uh_t.T, ct_t.T, ch_t.T, a_true, pa_t.T, ah_t.T)


# speedup vs baseline: 4.0225x; 1.2634x over previous
import numpy as np
import jax
import jax.numpy as jnp
from jax.experimental import pallas as pl

_K = 128
_N = 256
_HIDDEN = 512
_B = 1024
_THRESH = 0.5
_RATE = _K / _N


def _build_info_mask():
    m = int(np.log2(_N))
    z = np.array([0.5], dtype=np.float64)
    for _ in range(m):
        z = np.concatenate([2.0 * z - z * z, z * z])
    order = np.argsort(z, kind="stable")
    mask = np.zeros(_N, dtype=bool)
    mask[order[:_K]] = True
    return mask


_INFO_MASK = _build_info_mask()
_FROZEN = ~_INFO_MASK
_INFO_IDX = np.where(_INFO_MASK)[0]


def _encode_rows(mat):
    n = mat.shape[1]
    if n == 1:
        return mat
    h = n // 2
    a = _encode_rows(mat[:, :h])
    b = _encode_rows(mat[:, h:])
    return np.concatenate([a ^ b, b], axis=1)


_G = _encode_rows(np.eye(_N, dtype=np.int64))
_G_INFO = _G[_INFO_IDX, :].astype(np.float32)
_G_UHAT = _G[:, _INFO_IDX].astype(np.float32)


def _mod2(v):
    return v - 2.0 * jnp.floor(v * 0.5)


def _dot_t(a, b):
    return jax.lax.dot_general(a, b, (((0,), (0,)), ((), ())),
                               preferred_element_type=jnp.float32)


def _decode_x(llr, fr):
    if fr.all():
        return jnp.zeros_like(llr)
    if not fr.any():
        return (llr < 0.0).astype(llr.dtype)
    h = fr.shape[0] // 2
    la, lb = llr[:h], llr[h:]
    lf, rf = fr[:h], fr[h:]
    if lf.all():
        x2 = _decode_x(la + lb, rf)
        return jnp.concatenate([x2, x2], axis=0)
    l1 = jnp.sign(la) * jnp.sign(lb) * jnp.minimum(jnp.abs(la), jnp.abs(lb))
    x1 = _decode_x(l1, lf)
    l2 = lb + (1.0 - 2.0 * x1) * la
    x2 = _decode_x(l2, rf)
    return jnp.concatenate([x1 + x2 - 2.0 * x1 * x2, x2], axis=0)


def _fused_body(u_ref, a_ref, nr_ref, ni_ref, no_ref, s_ref,
                w1_ref, b1_ref, w2x_ref, b2x_ref, w2p_ref, b2p_ref,
                gi_ref, gu_ref, id_ref,
                ct_ref, uh_ref, ch_ref, pa_ref, ah_ref):
    ct = jax.lax.dot_general(u_ref[...], gi_ref[...], (((1,), (0,)), ((), ())),
                             preferred_element_type=jnp.float32)
    ct = _mod2(ct)
    ct_ref[...] = ct

    s = s_ref[0, 0]
    y_r = (1.0 - 2.0 * ct) * a_ref[...] + nr_ref[...] * s
    y_i = ni_ref[...] * s
    y_ri = jnp.concatenate([y_r, y_i], axis=1)

    h = jnp.maximum(
        jax.lax.dot_general(y_ri, w1_ref[...], (((1,), (0,)), ((), ())),
                            preferred_element_type=jnp.float32) + b1_ref[...],
        0.0)
    logit = jax.lax.dot_general(h, w2p_ref[...], (((1,), (0,)), ((), ())),
                                preferred_element_type=jnp.float32) + b2p_ref[...]
    p = jax.nn.sigmoid(logit)
    pa_ref[...] = p
    ah = (p > _THRESH).astype(jnp.float32)
    ah_ref[...] = ah

    yhr_t = jax.lax.dot_general(w2x_ref[...][:, :_N], h, (((0,), (1,)), ((), ())),
                                preferred_element_type=jnp.float32)
    yhr_t = yhr_t + b2x_ref[0:_N]
    llr_t = 4.0 * yhr_t / no_ref[0, 0]

    x_root = _decode_x(llr_t, _FROZEN)

    uh = _mod2(jax.lax.dot_general(x_root, gu_ref[...], (((0,), (0,)), ((), ())),
                                   preferred_element_type=jnp.float32))
    uh_ref[...] = uh * ah
    ch = jax.lax.dot_general(x_root, id_ref[...], (((0,), (0,)), ((), ())),
                             preferred_element_type=jnp.float32)
    ch_ref[...] = ch * ah


def kernel(u, a_true, noise_r, noise_i, ebno_db, W1, b1, W2x, b2x, W2p, b2p):
    no = 1.0 / (jnp.power(10.0, ebno_db / 10.0) * _RATE)
    s = jnp.sqrt(no / 2.0)
    out_shapes = [
        jax.ShapeDtypeStruct((_B, _N), jnp.float32),
        jax.ShapeDtypeStruct((_B, _K), jnp.float32),
        jax.ShapeDtypeStruct((_B, _N), jnp.float32),
        jax.ShapeDtypeStruct((_B, 1), jnp.float32),
        jax.ShapeDtypeStruct((_B, 1), jnp.float32),
    ]
    ct, uh, ch, pa, ah = pl.pallas_call(
        _fused_body,
        out_shape=out_shapes,
    )(
        u,
        a_true,
        noise_r,
        noise_i,
        no.reshape(1, 1),
        s.reshape(1, 1),
        W1,
        b1.reshape(1, _HIDDEN),
        W2x,
        b2x.reshape(2 * _N, 1),
        W2p,
        b2p.reshape(1, 1),
        jnp.asarray(_G_INFO),
        jnp.asarray(_G_UHAT),
        jnp.eye(_N, dtype=jnp.float32),
    )
    return (u, uh, ct, ch, a_true, pa, ah)
